# baseline (speedup 1.0000x reference)
import jax
import jax.numpy as jnp
from jax.experimental import pallas as pl
from jax.experimental.pallas import tpu as pltpu

VOCAB = 100000
D = 16
NQ = 1280
SUB = 4176
NSUB = 8
CHUNK = SUB * NSUB
NCHUNK = 3


def _knn_body(q_ref, e_ref, n_ref, val_ref, idx_ref, cval, cidx):
    i = pl.program_id(0)
    j = pl.program_id(1)
    e = e_ref[...]
    q_bf = q_ref[...]
    recip = 1.0 / n_ref[...]
    e_n = e * recip
    dn = (((1,), (1,)), ((), ()))
    sims = jax.lax.dot_general(e_n, q_bf, dn,
                               preferred_element_type=jnp.float32)
    dist = 1.0 - sims
    base = i * CHUNK + j * SUB
    row = jax.lax.broadcasted_iota(jnp.int32, dist.shape, 0)
    dist = jnp.where(row + base < VOCAB, dist, jnp.float32(jnp.inf))
    bmin = jnp.min(dist, axis=0)
    barg = jnp.min(jnp.where(dist == bmin[None, :], row, VOCAB), axis=0) + base

    @pl.when(j == 0)
    def _():
        cval[...] = bmin
        cidx[...] = barg

    @pl.when(j > 0)
    def _():
        b = bmin < cval[...]
        cval[...] = jnp.where(b, bmin, cval[...])
        cidx[...] = jnp.where(b, barg, cidx[...])

    @pl.when(j == NSUB - 1)
    def _():
        cv = cval[...]
        ci = cidx[...]

        @pl.when(i == 0)
        def _():
            val_ref[...] = cv.astype(jnp.bfloat16).astype(jnp.float32)
            idx_ref[...] = ci

        @pl.when(i > 0)
        def _():
            gb = cv < val_ref[...]
            nv = jnp.where(gb, cv, val_ref[...])
            val_ref[...] = nv.astype(jnp.bfloat16).astype(jnp.float32)
            idx_ref[...] = jnp.where(gb, ci, idx_ref[...])


def kernel(batch, emb_array):
    b, s, d = batch.shape
    nq = b * s
    bnorm = jnp.linalg.norm(batch, axis=2)
    q_bf = (batch / bnorm[:, :, None]).astype(jnp.bfloat16).reshape(nq, d)
    enorm = jnp.linalg.norm(emb_array, axis=1).reshape(VOCAB, 1)
    _, idx = pl.pallas_call(
        _knn_body,
        grid=(NCHUNK, NSUB),
        in_specs=[
            pl.BlockSpec((nq, d), lambda i, j: (0, 0)),
            pl.BlockSpec((SUB, d), lambda i, j: (i * NSUB + j, 0)),
            pl.BlockSpec((SUB, 1), lambda i, j: (i * NSUB + j, 0)),
        ],
        out_specs=[
            pl.BlockSpec((nq,), lambda i, j: (0,)),
            pl.BlockSpec((nq,), lambda i, j: (0,)),
        ],
        out_shape=[
            jax.ShapeDtypeStruct((nq,), jnp.float32),
            jax.ShapeDtypeStruct((nq,), jnp.int32),
        ],
        scratch_shapes=[
            pltpu.VMEM((nq,), jnp.float32),
            pltpu.VMEM((nq,), jnp.int32),
        ],
    )(q_bf, emb_array, enorm)
    return idx.reshape(b, s)
